# baseline (device time: 44417 ns/iter reference)
import jax
import jax.numpy as jnp
from jax import lax
from jax.experimental import pallas as pl
from jax.experimental.pallas import tpu as pltpu

N_DEV = 16
CAPACITY = 102.0
P = 128
CH = 32
N_CH = P // CH


def kernel(x, router_W, route_idx, expert_W):
    T, D = x.shape
    E_loc, _, H = expert_W.shape
    E = N_DEV * E_loc

    route_row = route_idx.reshape(1, T).astype(jnp.float32)

    def body(x_ref, rw_ref, rid_ref, w_ref, out_ref,
             idxbuf, sx, rx, sy, ry,
             i_send, i_recv, xp_send, xp_recv, y_send, y_recv):
        my = lax.axis_index("i")
        myf = my.astype(jnp.float32)

        idxbuf[pl.ds(my, 1), :] = rid_ref[...]
        for d in range(N_DEV):
            @pl.when(my != d)
            def _():
                pltpu.make_async_remote_copy(
                    src_ref=idxbuf.at[pl.ds(my, 1)],
                    dst_ref=idxbuf.at[pl.ds(my, 1)],
                    send_sem=i_send.at[d], recv_sem=i_recv.at[my],
                    device_id=(d,), device_id_type=pl.DeviceIdType.MESH,
                ).start()
        for s in range(N_DEV):
            @pl.when(my != s)
            def _():
                pltpu.make_async_remote_copy(
                    src_ref=idxbuf.at[pl.ds(s, 1)],
                    dst_ref=idxbuf.at[pl.ds(s, 1)],
                    send_sem=i_send.at[s], recv_sem=i_recv.at[s],
                    device_id=(s,), device_id_type=pl.DeviceIdType.MESH,
                ).wait_recv()

        ei = lax.broadcasted_iota(jnp.int32, (E, 1), 0)
        ei_f = ei.astype(jnp.float32)
        ei_blk = ei // E_loc
        triU = (lax.broadcasted_iota(jnp.int32, (T, T), 0)
                < lax.broadcasted_iota(jnp.int32, (T, T), 1)
                ).astype(jnp.bfloat16)
        qio = lax.broadcasted_iota(jnp.int32, (P, 1), 0
                                   ).astype(jnp.float32)
        zb = jnp.zeros((), jnp.bfloat16)

        hist_cols = []
        for s in range(N_DEV):
            er = idxbuf[pl.ds(s, 1), :]
            hist_cols.append(jnp.sum(
                (ei_f == er).astype(jnp.float32), axis=1, keepdims=True))
        c_cols = []
        acc = jnp.zeros((E, 1), jnp.float32)
        for s in range(N_DEV):
            c_cols.append(jnp.clip(CAPACITY - acc, 0.0, hist_cols[s]))
            acc = acc + hist_cols[s]

        er_m = rid_ref[...]
        c_mine = jnp.zeros((E, 1), jnp.float32)
        for s in range(N_DEV):
            c_mine = c_mine + jnp.where(my == s, c_cols[s], 0.0)
        ohT = (ei_f == er_m).astype(jnp.bfloat16)
        ranksT = jnp.dot(ohT, triU, preferred_element_type=jnp.float32)
        ohTf = ohT.astype(jnp.float32)
        r_m = jnp.sum(ohTf * ranksT, axis=0, keepdims=True)
        cnt_m = jnp.sum(ohTf * c_mine, axis=0, keepdims=True)
        blk_m = jnp.floor(er_m / E_loc)
        accept_m = r_m < cnt_m

        x_bf = x_ref[...].astype(jnp.bfloat16)
        Rts = []
        n_out = []
        lt = ei_f < er_m
        for c in range(N_DEV):
            in_c = ei_blk == c
            off_c = jnp.sum(jnp.where(in_c & lt, c_mine, 0.0),
                            axis=0, keepdims=True)
            valid = (blk_m == float(c)) & accept_m
            Rt = ((qio == off_c + r_m) & valid).astype(jnp.bfloat16)
            Rts.append(Rt)
            n_out.append(jnp.sum(jnp.where(in_c, c_mine, 0.0)))
        RtAll = jnp.concatenate(Rts, axis=0)
        xpAll = jnp.dot(RtAll, x_bf,
                        preferred_element_type=jnp.float32
                        ).astype(jnp.bfloat16)
        sx[...] = xpAll.reshape(N_DEV, P, D)
        for c in range(N_DEV):
            @pl.when(my == c)
            def _():
                rx[pl.ds(c, 1)] = sx[pl.ds(c, 1)]

            for k in range(N_CH):
                @pl.when((my != c) & (n_out[c] > float(CH * k)))
                def _():
                    pltpu.make_async_remote_copy(
                        src_ref=sx.at[c, pl.ds(CH * k, CH)],
                        dst_ref=rx.at[my, pl.ds(CH * k, CH)],
                        send_sem=xp_send.at[c, k],
                        recv_sem=xp_recv.at[my, k],
                        device_id=(c,),
                        device_id_type=pl.DeviceIdType.MESH,
                    ).start()

        w4 = w_ref[...].reshape(E_loc * D, H).astype(jnp.bfloat16)
        n_in = []
        for d in range(N_DEV):
            n_in.append(jnp.sum(
                jnp.where(ei_blk.astype(jnp.float32) == myf,
                          c_cols[d], 0.0)))
            for k in range(N_CH):
                @pl.when((my != d) & (n_in[d] > float(CH * k)))
                def _():
                    pltpu.make_async_remote_copy(
                        src_ref=rx.at[d, pl.ds(CH * k, CH)],
                        dst_ref=rx.at[d, pl.ds(CH * k, CH)],
                        send_sem=xp_send.at[d, k],
                        recv_sem=xp_recv.at[d, k],
                        device_id=(d,),
                        device_id_type=pl.DeviceIdType.MESH,
                    ).wait_recv()
        blocks = []
        for d in range(N_DEV):
            xp = rx[d]
            b_lo = jnp.zeros((), jnp.float32)
            parts = []
            for j in range(E_loc):
                mj = ei_f == (my * E_loc + j).astype(jnp.float32)
                b_hi = b_lo + jnp.sum(jnp.where(mj, c_cols[d], 0.0))
                parts.append(jnp.where((qio >= b_lo) & (qio < b_hi),
                                       xp, zb))
                b_lo = b_hi
            blocks.append(jnp.concatenate(parts, axis=1))
        xp4All = jnp.concatenate(blocks, axis=0)
        ypAll = jnp.dot(xp4All, w4, preferred_element_type=jnp.float32
                        ).astype(jnp.bfloat16)
        sy[...] = ypAll.reshape(N_DEV, P, H)
        for d in range(N_DEV):
            @pl.when(my == d)
            def _():
                ry[pl.ds(d, 1)] = sy[pl.ds(d, 1)]

            for k in range(N_CH):
                @pl.when((my != d) & (n_in[d] > float(CH * k)))
                def _():
                    pltpu.make_async_remote_copy(
                        src_ref=sy.at[d, pl.ds(CH * k, CH)],
                        dst_ref=ry.at[my, pl.ds(CH * k, CH)],
                        send_sem=y_send.at[d, k],
                        recv_sem=y_recv.at[my, k],
                        device_id=(d,),
                        device_id_type=pl.DeviceIdType.MESH,
                    ).start()

        for c in range(N_DEV):
            for k in range(N_CH):
                @pl.when((my != c) & (n_out[c] <= float(CH * k)))
                def _():
                    ry[c, pl.ds(CH * k, CH)] = jnp.zeros(
                        (CH, H), jnp.bfloat16)

                @pl.when((my != c) & (n_out[c] > float(CH * k)))
                def _():
                    pltpu.make_async_remote_copy(
                        src_ref=ry.at[c, pl.ds(CH * k, CH)],
                        dst_ref=ry.at[c, pl.ds(CH * k, CH)],
                        send_sem=y_send.at[c, k],
                        recv_sem=y_recv.at[c, k],
                        device_id=(c,),
                        device_id_type=pl.DeviceIdType.MESH,
                    ).wait_recv()
        yAll = ry[...].reshape(N_DEV * P, H)
        out_ref[...] = lax.dot_general(
            RtAll, yAll,
            dimension_numbers=(((0,), (0,)), ((), ())),
            preferred_element_type=jnp.float32)

        for d in range(N_DEV):
            @pl.when(my != d)
            def _():
                pltpu.make_async_remote_copy(
                    src_ref=idxbuf.at[pl.ds(my, 1)],
                    dst_ref=idxbuf.at[pl.ds(my, 1)],
                    send_sem=i_send.at[d], recv_sem=i_recv.at[my],
                    device_id=(d,), device_id_type=pl.DeviceIdType.MESH,
                ).wait_send()
            for k in range(N_CH):
                @pl.when((my != d) & (n_out[d] > float(CH * k)))
                def _():
                    pltpu.make_async_remote_copy(
                        src_ref=sx.at[d, pl.ds(CH * k, CH)],
                        dst_ref=rx.at[my, pl.ds(CH * k, CH)],
                        send_sem=xp_send.at[d, k],
                        recv_sem=xp_recv.at[my, k],
                        device_id=(d,),
                        device_id_type=pl.DeviceIdType.MESH,
                    ).wait_send()

                @pl.when((my != d) & (n_in[d] > float(CH * k)))
                def _():
                    pltpu.make_async_remote_copy(
                        src_ref=sy.at[d, pl.ds(CH * k, CH)],
                        dst_ref=ry.at[my, pl.ds(CH * k, CH)],
                        send_sem=y_send.at[d, k],
                        recv_sem=y_recv.at[my, k],
                        device_id=(d,),
                        device_id_type=pl.DeviceIdType.MESH,
                    ).wait_send()

    return pl.pallas_call(
        body,
        out_shape=jax.ShapeDtypeStruct((T, H), jnp.float32),
        in_specs=[pl.BlockSpec(memory_space=pltpu.VMEM)] * 4,
        out_specs=pl.BlockSpec(memory_space=pltpu.VMEM),
        scratch_shapes=[
            pltpu.VMEM((N_DEV, T), jnp.float32),
            pltpu.VMEM((N_DEV, P, D), jnp.bfloat16),
            pltpu.VMEM((N_DEV, P, D), jnp.bfloat16),
            pltpu.VMEM((N_DEV, P, H), jnp.bfloat16),
            pltpu.VMEM((N_DEV, P, H), jnp.bfloat16),
            pltpu.SemaphoreType.DMA((N_DEV,)),
            pltpu.SemaphoreType.DMA((N_DEV,)),
            pltpu.SemaphoreType.DMA((N_DEV, N_CH)),
            pltpu.SemaphoreType.DMA((N_DEV, N_CH)),
            pltpu.SemaphoreType.DMA((N_DEV, N_CH)),
            pltpu.SemaphoreType.DMA((N_DEV, N_CH)),
        ],
        compiler_params=pltpu.CompilerParams(
            vmem_limit_bytes=100 * 1024 * 1024,
        ),
    )(x, router_W, route_row, expert_W)


# device time: 41328 ns/iter; 1.0747x vs baseline; 1.0747x over previous
import jax
import jax.numpy as jnp
from jax import lax
from jax.experimental import pallas as pl
from jax.experimental.pallas import tpu as pltpu

N_DEV = 16
CAPACITY = 102.0
P = 128
CH = 32
N_CH = P // CH


def kernel(x, router_W, route_idx, expert_W):
    T, D = x.shape
    E_loc, _, H = expert_W.shape
    E = N_DEV * E_loc

    route_row = route_idx.reshape(1, T).astype(jnp.float32)

    def body(x_ref, rw_ref, rid_ref, w_ref, out_ref,
             idxbuf, sx, rx, sy, ry,
             i_send, i_recv, xp_send, xp_recv, y_send, y_recv):
        my = lax.axis_index("i")
        myf = my.astype(jnp.float32)

        idxbuf[pl.ds(my, 1), :] = rid_ref[...]
        for d in range(N_DEV):
            @pl.when(my != d)
            def _():
                pltpu.make_async_remote_copy(
                    src_ref=idxbuf.at[pl.ds(my, 1)],
                    dst_ref=idxbuf.at[pl.ds(my, 1)],
                    send_sem=i_send.at[d], recv_sem=i_recv.at[my],
                    device_id=(d,), device_id_type=pl.DeviceIdType.MESH,
                ).start()

        ei = lax.broadcasted_iota(jnp.int32, (E, 1), 0)
        ei_f = ei.astype(jnp.float32)
        ei_blk = ei // E_loc
        triU = (lax.broadcasted_iota(jnp.int32, (T, T), 0)
                < lax.broadcasted_iota(jnp.int32, (T, T), 1)
                ).astype(jnp.bfloat16)
        qio = lax.broadcasted_iota(jnp.int32, (P, 1), 0
                                   ).astype(jnp.float32)
        zb = jnp.zeros((), jnp.bfloat16)

        er_m = rid_ref[...]
        ohT = (ei_f == er_m).astype(jnp.bfloat16)
        ranksT = jnp.dot(ohT, triU, preferred_element_type=jnp.float32)
        ohTf = ohT.astype(jnp.float32)
        r_m = jnp.sum(ohTf * ranksT, axis=0, keepdims=True)
        h_loc = jnp.sum(ohTf, axis=1, keepdims=True)
        blk_m = jnp.floor(er_m / E_loc)
        lt = ei_f < er_m

        x_bf = x_ref[...].astype(jnp.bfloat16)
        Rts = []
        n_disp = []
        for c in range(N_DEV):
            in_c = ei_blk == c
            off_c = jnp.sum(jnp.where(in_c & lt, h_loc, 0.0),
                            axis=0, keepdims=True)
            valid = blk_m == float(c)
            Rt = ((qio == off_c + r_m) & valid).astype(jnp.bfloat16)
            Rts.append(Rt)
            n_disp.append(jnp.sum(jnp.where(in_c, h_loc, 0.0)))
        RtAll = jnp.concatenate(Rts, axis=0)
        xpAll = jnp.dot(RtAll, x_bf,
                        preferred_element_type=jnp.float32
                        ).astype(jnp.bfloat16)
        sx[...] = xpAll.reshape(N_DEV, P, D)
        for c in range(N_DEV):
            @pl.when(my == c)
            def _():
                rx[pl.ds(c, 1)] = sx[pl.ds(c, 1)]

            for k in range(N_CH):
                @pl.when((my != c) & (n_disp[c] > float(CH * k)))
                def _():
                    pltpu.make_async_remote_copy(
                        src_ref=sx.at[c, pl.ds(CH * k, CH)],
                        dst_ref=rx.at[my, pl.ds(CH * k, CH)],
                        send_sem=xp_send.at[c, k],
                        recv_sem=xp_recv.at[my, k],
                        device_id=(c,),
                        device_id_type=pl.DeviceIdType.MESH,
                    ).start()

        for s in range(N_DEV):
            @pl.when(my != s)
            def _():
                pltpu.make_async_remote_copy(
                    src_ref=idxbuf.at[pl.ds(s, 1)],
                    dst_ref=idxbuf.at[pl.ds(s, 1)],
                    send_sem=i_send.at[s], recv_sem=i_recv.at[s],
                    device_id=(s,), device_id_type=pl.DeviceIdType.MESH,
                ).wait_recv()
        hist_cols = []
        for s in range(N_DEV):
            er = idxbuf[pl.ds(s, 1), :]
            hist_cols.append(jnp.sum(
                (ei_f == er).astype(jnp.float32), axis=1, keepdims=True))
        c_cols = []
        acc = jnp.zeros((E, 1), jnp.float32)
        for s in range(N_DEV):
            c_cols.append(jnp.clip(CAPACITY - acc, 0.0, hist_cols[s]))
            acc = acc + hist_cols[s]

        w4 = w_ref[...].reshape(E_loc * D, H).astype(jnp.bfloat16)
        n_in = []
        for d in range(N_DEV):
            n_in.append(jnp.sum(
                jnp.where(ei_blk.astype(jnp.float32) == myf,
                          hist_cols[d], 0.0)))
            for k in range(N_CH):
                @pl.when((my != d) & (n_in[d] > float(CH * k)))
                def _():
                    pltpu.make_async_remote_copy(
                        src_ref=rx.at[d, pl.ds(CH * k, CH)],
                        dst_ref=rx.at[d, pl.ds(CH * k, CH)],
                        send_sem=xp_send.at[d, k],
                        recv_sem=xp_recv.at[d, k],
                        device_id=(d,),
                        device_id_type=pl.DeviceIdType.MESH,
                    ).wait_recv()
        blocks = []
        for d in range(N_DEV):
            xp = rx[d]
            hb_lo = jnp.zeros((), jnp.float32)
            parts = []
            for j in range(E_loc):
                mj = ei_f == (my * E_loc + j).astype(jnp.float32)
                hj = jnp.sum(jnp.where(mj, hist_cols[d], 0.0))
                cj = jnp.sum(jnp.where(mj, c_cols[d], 0.0))
                parts.append(jnp.where(
                    (qio >= hb_lo) & (qio < hb_lo + cj), xp, zb))
                hb_lo = hb_lo + hj
            blocks.append(jnp.concatenate(parts, axis=1))
        xp4All = jnp.concatenate(blocks, axis=0)
        ypAll = jnp.dot(xp4All, w4, preferred_element_type=jnp.float32
                        ).astype(jnp.bfloat16)
        sy[...] = ypAll.reshape(N_DEV, P, H)
        for d in range(N_DEV):
            @pl.when(my == d)
            def _():
                ry[pl.ds(d, 1)] = sy[pl.ds(d, 1)]

            for k in range(N_CH):
                @pl.when((my != d) & (n_in[d] > float(CH * k)))
                def _():
                    pltpu.make_async_remote_copy(
                        src_ref=sy.at[d, pl.ds(CH * k, CH)],
                        dst_ref=ry.at[my, pl.ds(CH * k, CH)],
                        send_sem=y_send.at[d, k],
                        recv_sem=y_recv.at[my, k],
                        device_id=(d,),
                        device_id_type=pl.DeviceIdType.MESH,
                    ).start()

        for c in range(N_DEV):
            for k in range(N_CH):
                @pl.when((my != c) & (n_disp[c] <= float(CH * k)))
                def _():
                    ry[c, pl.ds(CH * k, CH)] = jnp.zeros(
                        (CH, H), jnp.bfloat16)

                @pl.when((my != c) & (n_disp[c] > float(CH * k)))
                def _():
                    pltpu.make_async_remote_copy(
                        src_ref=ry.at[c, pl.ds(CH * k, CH)],
                        dst_ref=ry.at[c, pl.ds(CH * k, CH)],
                        send_sem=y_send.at[c, k],
                        recv_sem=y_recv.at[c, k],
                        device_id=(c,),
                        device_id_type=pl.DeviceIdType.MESH,
                    ).wait_recv()
        yAll = ry[...].reshape(N_DEV * P, H)
        out_ref[...] = lax.dot_general(
            RtAll, yAll,
            dimension_numbers=(((0,), (0,)), ((), ())),
            preferred_element_type=jnp.float32)

        for d in range(N_DEV):
            @pl.when(my != d)
            def _():
                pltpu.make_async_remote_copy(
                    src_ref=idxbuf.at[pl.ds(my, 1)],
                    dst_ref=idxbuf.at[pl.ds(my, 1)],
                    send_sem=i_send.at[d], recv_sem=i_recv.at[my],
                    device_id=(d,), device_id_type=pl.DeviceIdType.MESH,
                ).wait_send()
            for k in range(N_CH):
                @pl.when((my != d) & (n_disp[d] > float(CH * k)))
                def _():
                    pltpu.make_async_remote_copy(
                        src_ref=sx.at[d, pl.ds(CH * k, CH)],
                        dst_ref=rx.at[my, pl.ds(CH * k, CH)],
                        send_sem=xp_send.at[d, k],
                        recv_sem=xp_recv.at[my, k],
                        device_id=(d,),
                        device_id_type=pl.DeviceIdType.MESH,
                    ).wait_send()

                @pl.when((my != d) & (n_in[d] > float(CH * k)))
                def _():
                    pltpu.make_async_remote_copy(
                        src_ref=sy.at[d, pl.ds(CH * k, CH)],
                        dst_ref=ry.at[my, pl.ds(CH * k, CH)],
                        send_sem=y_send.at[d, k],
                        recv_sem=y_recv.at[my, k],
                        device_id=(d,),
                        device_id_type=pl.DeviceIdType.MESH,
                    ).wait_send()

    return pl.pallas_call(
        body,
        out_shape=jax.ShapeDtypeStruct((T, H), jnp.float32),
        in_specs=[pl.BlockSpec(memory_space=pltpu.VMEM)] * 4,
        out_specs=pl.BlockSpec(memory_space=pltpu.VMEM),
        scratch_shapes=[
            pltpu.VMEM((N_DEV, T), jnp.float32),
            pltpu.VMEM((N_DEV, P, D), jnp.bfloat16),
            pltpu.VMEM((N_DEV, P, D), jnp.bfloat16),
            pltpu.VMEM((N_DEV, P, H), jnp.bfloat16),
            pltpu.VMEM((N_DEV, P, H), jnp.bfloat16),
            pltpu.SemaphoreType.DMA((N_DEV,)),
            pltpu.SemaphoreType.DMA((N_DEV,)),
            pltpu.SemaphoreType.DMA((N_DEV, N_CH)),
            pltpu.SemaphoreType.DMA((N_DEV, N_CH)),
            pltpu.SemaphoreType.DMA((N_DEV, N_CH)),
            pltpu.SemaphoreType.DMA((N_DEV, N_CH)),
        ],
        compiler_params=pltpu.CompilerParams(
            vmem_limit_bytes=100 * 1024 * 1024,
        ),
    )(x, router_W, route_row, expert_W)


# device time: 35824 ns/iter; 1.2399x vs baseline; 1.1536x over previous
import jax
import jax.numpy as jnp
from jax import lax
from jax.experimental import pallas as pl
from jax.experimental.pallas import tpu as pltpu

N_DEV = 16
CAPACITY = 102.0
P = 96
CH = 32
N_CH = P // CH


def kernel(x, router_W, route_idx, expert_W):
    T, D = x.shape
    E_loc, _, H = expert_W.shape
    E = N_DEV * E_loc

    route_row = route_idx.reshape(1, T).astype(jnp.float32)

    def body(x_ref, rw_ref, rid_ref, w_ref, out_ref,
             idxbuf, sx, rx, sy, ry,
             i_send, i_recv, xp_send, xp_recv, y_send, y_recv):
        my = lax.axis_index("i")
        myf = my.astype(jnp.float32)

        barrier_sem = pltpu.get_barrier_semaphore()
        for d in range(N_DEV):
            @pl.when(my != d)
            def _():
                pl.semaphore_signal(
                    barrier_sem, inc=1, device_id=(d,),
                    device_id_type=pl.DeviceIdType.MESH)
        pl.semaphore_wait(barrier_sem, N_DEV - 1)

        idxbuf[pl.ds(my, 1), :] = rid_ref[...]
        for d in range(N_DEV):
            @pl.when(my != d)
            def _():
                pltpu.make_async_remote_copy(
                    src_ref=idxbuf.at[pl.ds(my, 1)],
                    dst_ref=idxbuf.at[pl.ds(my, 1)],
                    send_sem=i_send.at[d], recv_sem=i_recv.at[my],
                    device_id=(d,), device_id_type=pl.DeviceIdType.MESH,
                ).start()

        ei = lax.broadcasted_iota(jnp.int32, (E, 1), 0)
        ei_f = ei.astype(jnp.float32)
        ei_blk = ei // E_loc
        triU = (lax.broadcasted_iota(jnp.int32, (T, T), 0)
                < lax.broadcasted_iota(jnp.int32, (T, T), 1)
                ).astype(jnp.bfloat16)
        qio = lax.broadcasted_iota(jnp.int32, (P, 1), 0
                                   ).astype(jnp.float32)
        zb = jnp.zeros((), jnp.bfloat16)

        er_m = rid_ref[...]
        ohT = (ei_f == er_m).astype(jnp.bfloat16)
        ranksT = jnp.dot(ohT, triU, preferred_element_type=jnp.float32)
        ohTf = ohT.astype(jnp.float32)
        r_m = jnp.sum(ohTf * ranksT, axis=0, keepdims=True)
        h_loc = jnp.sum(ohTf, axis=1, keepdims=True)
        blk_m = jnp.floor(er_m / E_loc)
        lt = ei_f < er_m

        x_bf = x_ref[...].astype(jnp.bfloat16)
        Rts = []
        n_disp = []
        for c in range(N_DEV):
            in_c = ei_blk == c
            off_c = jnp.sum(jnp.where(in_c & lt, h_loc, 0.0),
                            axis=0, keepdims=True)
            valid = blk_m == float(c)
            Rt = ((qio == off_c + r_m) & valid).astype(jnp.bfloat16)
            Rts.append(Rt)
            n_disp.append(jnp.sum(jnp.where(in_c, h_loc, 0.0)))
        RtAll = jnp.concatenate(Rts, axis=0)
        xpAll = jnp.dot(RtAll, x_bf,
                        preferred_element_type=jnp.float32
                        ).astype(jnp.bfloat16)
        sx[...] = xpAll.reshape(N_DEV, P, D)
        for c in range(N_DEV):
            @pl.when(my == c)
            def _():
                rx[pl.ds(c, 1)] = sx[pl.ds(c, 1)]

            for k in range(N_CH):
                @pl.when((my != c) & (n_disp[c] > float(CH * k)))
                def _():
                    pltpu.make_async_remote_copy(
                        src_ref=sx.at[c, pl.ds(CH * k, CH)],
                        dst_ref=rx.at[my, pl.ds(CH * k, CH)],
                        send_sem=xp_send.at[c, k],
                        recv_sem=xp_recv.at[my, k],
                        device_id=(c,),
                        device_id_type=pl.DeviceIdType.MESH,
                    ).start()

        for s in range(N_DEV):
            @pl.when(my != s)
            def _():
                pltpu.make_async_remote_copy(
                    src_ref=idxbuf.at[pl.ds(s, 1)],
                    dst_ref=idxbuf.at[pl.ds(s, 1)],
                    send_sem=i_send.at[s], recv_sem=i_recv.at[s],
                    device_id=(s,), device_id_type=pl.DeviceIdType.MESH,
                ).wait_recv()
        hist_cols = []
        for s in range(N_DEV):
            er = idxbuf[pl.ds(s, 1), :]
            hist_cols.append(jnp.sum(
                (ei_f == er).astype(jnp.float32), axis=1, keepdims=True))
        c_cols = []
        acc = jnp.zeros((E, 1), jnp.float32)
        for s in range(N_DEV):
            c_cols.append(jnp.clip(CAPACITY - acc, 0.0, hist_cols[s]))
            acc = acc + hist_cols[s]

        w4 = w_ref[...].reshape(E_loc * D, H).astype(jnp.bfloat16)
        n_in = []
        for d in range(N_DEV):
            n_in.append(jnp.sum(
                jnp.where(ei_blk.astype(jnp.float32) == myf,
                          hist_cols[d], 0.0)))
            for k in range(N_CH):
                @pl.when((my != d) & (n_in[d] > float(CH * k)))
                def _():
                    pltpu.make_async_remote_copy(
                        src_ref=rx.at[d, pl.ds(CH * k, CH)],
                        dst_ref=rx.at[d, pl.ds(CH * k, CH)],
                        send_sem=xp_send.at[d, k],
                        recv_sem=xp_recv.at[d, k],
                        device_id=(d,),
                        device_id_type=pl.DeviceIdType.MESH,
                    ).wait_recv()
        blocks = []
        for d in range(N_DEV):
            xp = rx[d]
            hb_lo = jnp.zeros((), jnp.float32)
            parts = []
            for j in range(E_loc):
                mj = ei_f == (my * E_loc + j).astype(jnp.float32)
                hj = jnp.sum(jnp.where(mj, hist_cols[d], 0.0))
                cj = jnp.sum(jnp.where(mj, c_cols[d], 0.0))
                parts.append(jnp.where(
                    (qio >= hb_lo) & (qio < hb_lo + cj), xp, zb))
                hb_lo = hb_lo + hj
            blocks.append(jnp.concatenate(parts, axis=1))
        xp4All = jnp.concatenate(blocks, axis=0)
        ypAll = jnp.dot(xp4All, w4, preferred_element_type=jnp.float32
                        ).astype(jnp.bfloat16)
        sy[...] = ypAll.reshape(N_DEV, P, H)
        for d in range(N_DEV):
            @pl.when(my == d)
            def _():
                ry[pl.ds(d, 1)] = sy[pl.ds(d, 1)]

            for k in range(N_CH):
                @pl.when((my != d) & (n_in[d] > float(CH * k)))
                def _():
                    pltpu.make_async_remote_copy(
                        src_ref=sy.at[d, pl.ds(CH * k, CH)],
                        dst_ref=ry.at[my, pl.ds(CH * k, CH)],
                        send_sem=y_send.at[d, k],
                        recv_sem=y_recv.at[my, k],
                        device_id=(d,),
                        device_id_type=pl.DeviceIdType.MESH,
                    ).start()

        for c in range(N_DEV):
            for k in range(N_CH):
                @pl.when((my != c) & (n_disp[c] <= float(CH * k)))
                def _():
                    ry[c, pl.ds(CH * k, CH)] = jnp.zeros(
                        (CH, H), jnp.bfloat16)

                @pl.when((my != c) & (n_disp[c] > float(CH * k)))
                def _():
                    pltpu.make_async_remote_copy(
                        src_ref=ry.at[c, pl.ds(CH * k, CH)],
                        dst_ref=ry.at[c, pl.ds(CH * k, CH)],
                        send_sem=y_send.at[c, k],
                        recv_sem=y_recv.at[c, k],
                        device_id=(c,),
                        device_id_type=pl.DeviceIdType.MESH,
                    ).wait_recv()
        yAll = ry[...].reshape(N_DEV * P, H)
        out_ref[...] = lax.dot_general(
            RtAll, yAll,
            dimension_numbers=(((0,), (0,)), ((), ())),
            preferred_element_type=jnp.float32)

        for d in range(N_DEV):
            @pl.when(my != d)
            def _():
                pltpu.make_async_remote_copy(
                    src_ref=idxbuf.at[pl.ds(my, 1)],
                    dst_ref=idxbuf.at[pl.ds(my, 1)],
                    send_sem=i_send.at[d], recv_sem=i_recv.at[my],
                    device_id=(d,), device_id_type=pl.DeviceIdType.MESH,
                ).wait_send()
            for k in range(N_CH):
                @pl.when((my != d) & (n_disp[d] > float(CH * k)))
                def _():
                    pltpu.make_async_remote_copy(
                        src_ref=sx.at[d, pl.ds(CH * k, CH)],
                        dst_ref=rx.at[my, pl.ds(CH * k, CH)],
                        send_sem=xp_send.at[d, k],
                        recv_sem=xp_recv.at[my, k],
                        device_id=(d,),
                        device_id_type=pl.DeviceIdType.MESH,
                    ).wait_send()

                @pl.when((my != d) & (n_in[d] > float(CH * k)))
                def _():
                    pltpu.make_async_remote_copy(
                        src_ref=sy.at[d, pl.ds(CH * k, CH)],
                        dst_ref=ry.at[my, pl.ds(CH * k, CH)],
                        send_sem=y_send.at[d, k],
                        recv_sem=y_recv.at[my, k],
                        device_id=(d,),
                        device_id_type=pl.DeviceIdType.MESH,
                    ).wait_send()

    return pl.pallas_call(
        body,
        out_shape=jax.ShapeDtypeStruct((T, H), jnp.float32),
        in_specs=[pl.BlockSpec(memory_space=pltpu.VMEM)] * 4,
        out_specs=pl.BlockSpec(memory_space=pltpu.VMEM),
        scratch_shapes=[
            pltpu.VMEM((N_DEV, T), jnp.float32),
            pltpu.VMEM((N_DEV, P, D), jnp.bfloat16),
            pltpu.VMEM((N_DEV, P, D), jnp.bfloat16),
            pltpu.VMEM((N_DEV, P, H), jnp.bfloat16),
            pltpu.VMEM((N_DEV, P, H), jnp.bfloat16),
            pltpu.SemaphoreType.DMA((N_DEV,)),
            pltpu.SemaphoreType.DMA((N_DEV,)),
            pltpu.SemaphoreType.DMA((N_DEV, N_CH)),
            pltpu.SemaphoreType.DMA((N_DEV, N_CH)),
            pltpu.SemaphoreType.DMA((N_DEV, N_CH)),
            pltpu.SemaphoreType.DMA((N_DEV, N_CH)),
        ],
        compiler_params=pltpu.CompilerParams(
            vmem_limit_bytes=100 * 1024 * 1024,
            collective_id=0,
        ),
    )(x, router_W, route_row, expert_W)
